# baseline (speedup 1.0000x reference)
import functools

import jax
import jax.numpy as jnp
import numpy as np
from jax import lax
from jax.experimental import pallas as pl
from jax.experimental.pallas import tpu as pltpu
from jax.experimental.pallas import tpu_sc as plsc

N = 10000
E = 320000
NF = 128
NH = 128
ALPHA = 0.2
DAUG = NH + 16
NW = 32
EPW = E // NW
K = 80
NCH = EPW // K
ZR = 125
RPT = N // 16
ROWB = 400
NSUP = 5
SCH = NCH // NSUP
NPAIR = SCH // 2
DPK = 80

_SIGLO = np.array([32 * (i // 16) + i % 16 for i in range(NH // 2)],
                  dtype=np.int32)
_SIGHI = _SIGLO + 16


def _rne_bf16_bits(v):
    b = lax.bitcast_convert_type(v, jnp.uint32)
    return (b + jnp.uint32(0x7FFF) + ((b >> 16) & jnp.uint32(1))) >> 16


def _proj_body(x_ref, w_ref, wlo_ref, whi_ref, a_ref,
               haug_ref, s12_ref, hp_ref):
    h = jnp.dot(x_ref[...], w_ref[...], preferred_element_type=jnp.float32)
    pad = jnp.zeros((h.shape[0], DAUG - NH - 1), jnp.float32)
    ones = jnp.ones((h.shape[0], 1), jnp.float32)
    haug_ref[...] = jnp.concatenate([h, ones, pad], axis=1)
    a2d = jnp.concatenate([a_ref[:, :NH], a_ref[:, NH:]], axis=0)
    s12_ref[...] = lax.dot_general(a2d, h, (((1,), (1,)), ((), ())),
                                   preferred_element_type=jnp.float32)
    mlo = jnp.dot(x_ref[...], wlo_ref[...], preferred_element_type=jnp.float32)
    mhi = jnp.dot(x_ref[...], whi_ref[...], preferred_element_type=jnp.float32)
    nrow = mlo.shape[0]
    lo65 = jnp.concatenate([mlo, jnp.ones((nrow, 1), jnp.float32)], axis=1)
    hi65 = jnp.concatenate([mhi, jnp.zeros((nrow, 1), jnp.float32)], axis=1)
    word = (_rne_bf16_bits(hi65) << 16) | _rne_bf16_bits(lo65)
    packed = lax.bitcast_convert_type(word, jnp.float32)
    hp_ref[...] = jnp.concatenate(
        [packed, jnp.zeros((nrow, DPK - (NH + 2) // 2), jnp.float32)], axis=1)


_proj_call = pl.pallas_call(
    _proj_body,
    out_shape=[
        jax.ShapeDtypeStruct((N, DAUG), jnp.float32),
        jax.ShapeDtypeStruct((2, N), jnp.float32),
        jax.ShapeDtypeStruct((N, DPK), jnp.float32),
    ],
)


_BCAST_DNUMS = lax.GatherDimensionNumbers(
    offset_dims=(), collapsed_slice_dims=(0,), start_index_map=(0,))


def _lane_bcast(vec, r):
    idx = jnp.full((16, 1), r, jnp.int32)
    return lax.gather(vec, idx, _BCAST_DNUMS, (1,),
                      mode=lax.GatherScatterMode.PROMISE_IN_BOUNDS)


def _edge_body(src_hbm, dst_hbm, s1_hbm, s2_hbm, hp_hbm, out_hbm,
               src_s, dst_s, hbuf0, hbuf1, st0, st1,
               s1g0, s1g1, s2g0, s2g1,
               acc, gsem0, gsem1, ssem0, ssem1, zsem):
    cid = lax.axis_index("c")
    sid = lax.axis_index("s")
    wid = sid * 2 + cid
    hbuf = (hbuf0, hbuf1)
    st = (st0, st1)
    s1g = (s1g0, s1g1)
    s2g = (s2g0, s2g1)
    gsem = (gsem0, gsem1)
    ssem = (ssem0, ssem1)

    zeros16 = jnp.zeros((16,), jnp.float32)

    def zero_row(r, _):
        for j in range(DAUG // 16):
            st0[r, pl.ds(j * 16, 16)] = zeros16
        return 0

    lax.fori_loop(0, K, zero_row, 0)
    nz, rem = RPT // K, RPT - (RPT // K) * K
    for z in range(nz):
        pltpu.async_copy(st0, acc.at[pl.ds(sid * RPT + z * K, K)], zsem)
    pltpu.async_copy(st0.at[pl.ds(0, rem)],
                     acc.at[pl.ds(sid * RPT + nz * K, rem)], zsem)
    for z in range(nz):
        pltpu.make_async_copy(st0, acc.at[pl.ds(0, K)], zsem).wait()
    pltpu.make_async_copy(st0.at[pl.ds(0, rem)],
                          acc.at[pl.ds(0, rem)], zsem).wait()
    plsc.subcore_barrier()

    def issue_g(c, p):
        pltpu.async_copy(hp_hbm.at[dst_s.at[c]], hbuf[p], gsem[p])
        pltpu.async_copy(s1_hbm.at[src_s.at[c]], s1g[p], gsem[p])
        pltpu.async_copy(s2_hbm.at[dst_s.at[c]], s2g[p], gsem[p])

    def wait_g(p):
        pltpu.make_async_copy(hp_hbm.at[pl.ds(0, K)], hbuf[p], gsem[p]).wait()
        pltpu.make_async_copy(s1_hbm.at[pl.ds(0, K)], s1g[p], gsem[p]).wait()
        pltpu.make_async_copy(s2_hbm.at[pl.ds(0, K)], s2g[p], gsem[p]).wait()

    def wait_s(p):
        pltpu.make_async_copy(st[p], acc.at[pl.ds(0, K)], ssem[p]).wait()

    def section(c, p, wait_ss, issue_next):
        wait_g(p)

        if wait_ss is True:
            wait_s(p)
        elif wait_ss is not False:
            @pl.when(wait_ss)
            def _():
                wait_s(p)

        @plsc.parallel_loop(0, K, step=16)
        def mul_body(r0):
            sl0 = pl.ds(r0, 16)
            lg = s1g[p][sl0] + s2g[p][sl0]
            wv16 = jnp.exp(-jnp.maximum(lg, lg * ALPHA))
            for r in range(16):
                wvec = _lane_bcast(wv16, r)
                row = r0 + r
                for j in range(DPK // 16):
                    x16 = hbuf[p][row, pl.ds(16 * j, 16)]
                    va, vb = plsc.unpack(
                        plsc.bitcast(x16, jnp.bfloat16),
                        format=plsc.PackFormat.INTERLEAVED,
                        preferred_element_type=jnp.float32)
                    st[p][row, pl.ds(32 * j, 16)] = va * wvec
                    if j < 4:
                        st[p][row, pl.ds(32 * j + 16, 16)] = vb * wvec

        if issue_next is True:
            issue_g(c + 2, p)
        elif issue_next is not False:
            @pl.when(issue_next)
            def _():
                issue_g(c + 2, p)
        pltpu.async_copy(st[p], acc.at[src_s.at[c]], ssem[p], add=True)

    def super_body(u, _):
        pltpu.sync_copy(src_hbm.at[wid].at[pl.ds(u * SCH, SCH)], src_s)
        pltpu.sync_copy(dst_hbm.at[wid].at[pl.ds(u * SCH, SCH)], dst_s)
        issue_g(0, 0)
        issue_g(1, 1)

        def pair(t, _):
            section(2 * t, 0, wait_ss=(t > 0), issue_next=True)
            section(2 * t + 1, 1, wait_ss=(t > 0), issue_next=(t < NPAIR - 1))
            return 0

        lax.fori_loop(0, NPAIR, pair, 0)
        section(SCH - 1, 0, wait_ss=True, issue_next=False)
        wait_s(0)
        wait_s(1)
        return 0

    lax.fori_loop(0, NSUP, super_body, 0)

    plsc.subcore_barrier()
    pltpu.sync_copy(acc.at[pl.ds(sid * RPT, RPT)],
                    out_hbm.at[cid].at[pl.ds(sid * RPT, RPT)])


_edge_call = functools.partial(
    pl.kernel,
    out_type=jax.ShapeDtypeStruct((2, N, DAUG), jnp.float32),
    mesh=plsc.VectorSubcoreMesh(core_axis_name="c", subcore_axis_name="s"),
    compiler_params=pltpu.CompilerParams(use_tc_tiling_on_sc=False,
                                         needs_layout_passes=False),
    scratch_types=(
        [pltpu.VMEM((SCH, K), jnp.int32),
         pltpu.VMEM((SCH, K), jnp.int32)]
        + [pltpu.VMEM((K, DPK), jnp.float32)] * 2
        + [pltpu.VMEM((K, DAUG), jnp.float32)] * 2
        + [pltpu.VMEM((K,), jnp.float32)] * 4
        + [pltpu.VMEM_SHARED((N, DAUG), jnp.float32)]
        + [pltpu.SemaphoreType.DMA] * 5
    ),
)(_edge_body)


def _final_body(haug_ref, n0_ref, n1_ref, o_ref):
    ns = n0_ref[...] + n1_ref[...]
    hp = ns[:, :NH] / (ns[:, NH:NH + 1] + 1e-16)
    y = haug_ref[:, :NH] - hp
    o_ref[...] = jnp.where(y > 0, y, jnp.exp(y) - 1.0)


_final_call = pl.pallas_call(
    _final_body,
    grid=(N // ROWB,),
    in_specs=[
        pl.BlockSpec((ROWB, DAUG), lambda i: (i, 0)),
        pl.BlockSpec((ROWB, DAUG), lambda i: (i, 0)),
        pl.BlockSpec((ROWB, DAUG), lambda i: (i, 0)),
    ],
    out_specs=pl.BlockSpec((ROWB, NH), lambda i: (i, 0)),
    out_shape=jax.ShapeDtypeStruct((N, NH), jnp.float32),
)


def kernel(x, adj, no_need_param, W, a):
    src = adj[0].reshape(NW, NCH, K)
    dst = adj[1].reshape(NW, NCH, K)
    haug, s12, hp = _proj_call(x, W, W[:, _SIGLO], W[:, _SIGHI], a)
    part = _edge_call(src, dst, s12[0], s12[1], hp)
    return _final_call(haug, part[0], part[1])


# speedup vs baseline: 1.6616x; 1.6616x over previous
import functools

import jax
import jax.numpy as jnp
from jax import lax
from jax.experimental import pallas as pl
from jax.experimental.pallas import tpu as pltpu
from jax.experimental.pallas import tpu_sc as plsc

N = 10000
E = 320000
NF = 128
NH = 128
ALPHA = 0.2
DAUG = NH + 16
NW = 32
EPW = E // NW
K = 80
NCH = EPW // K
ZR = 125
RPT = N // 16
ROWB = 400


def _proj_body(x_ref, w_ref, a_ref, haug_ref, s12_ref):
    h = jnp.dot(x_ref[...], w_ref[...], preferred_element_type=jnp.float32)
    pad = jnp.zeros((h.shape[0], DAUG - NH - 2), jnp.float32)
    ones = jnp.ones((h.shape[0], 1), jnp.float32)
    a2d = jnp.concatenate([a_ref[:, :NH], a_ref[:, NH:]], axis=0)
    s12 = lax.dot_general(a2d, h, (((1,), (1,)), ((), ())),
                          preferred_element_type=jnp.float32)
    s12_ref[...] = s12
    s2col = lax.dot_general(h, a2d[1:2, :], (((1,), (1,)), ((), ())),
                            preferred_element_type=jnp.float32)
    haug_ref[...] = jnp.concatenate([h, ones, s2col, pad], axis=1)


_proj_call = pl.pallas_call(
    _proj_body,
    out_shape=[
        jax.ShapeDtypeStruct((N, DAUG), jnp.float32),
        jax.ShapeDtypeStruct((2, N), jnp.float32),
    ],
)


NSUP = 5
SCH = NCH // NSUP
NTRI = SCH // 3

_BCAST_DNUMS = lax.GatherDimensionNumbers(
    offset_dims=(), collapsed_slice_dims=(0,), start_index_map=(0,))


def _lane_bcast(vec, r):
    idx = jnp.full((16, 1), r, jnp.int32)
    return lax.gather(vec, idx, _BCAST_DNUMS, (1,),
                      mode=lax.GatherScatterMode.PROMISE_IN_BOUNDS)


def _edge_body(src_hbm, dst_hbm, s1_hbm, haug_hbm, out_hbm,
               src_s, dst_s,
               rows0, rows1, rows2, s1g0, s1g1, s1g2,
               acc, gsem0, gsem1, gsem2, ssem0, ssem1, ssem2, zsem):
    cid = lax.axis_index("c")
    sid = lax.axis_index("s")
    wid = sid * 2 + cid
    rows = (rows0, rows1, rows2)
    s1g = (s1g0, s1g1, s1g2)
    gsem = (gsem0, gsem1, gsem2)
    ssem = (ssem0, ssem1, ssem2)

    zeros16 = jnp.zeros((16,), jnp.float32)

    def zero_row(r, _):
        for j in range(DAUG // 16):
            rows0[r, pl.ds(j * 16, 16)] = zeros16
        return 0

    lax.fori_loop(0, K, zero_row, 0)
    nz, rem = RPT // K, RPT - (RPT // K) * K
    for z in range(nz):
        pltpu.async_copy(rows0, acc.at[pl.ds(sid * RPT + z * K, K)], zsem)
    pltpu.async_copy(rows0.at[pl.ds(0, rem)],
                     acc.at[pl.ds(sid * RPT + nz * K, rem)], zsem)
    for z in range(nz):
        pltpu.make_async_copy(rows0, acc.at[pl.ds(0, K)], zsem).wait()
    pltpu.make_async_copy(rows0.at[pl.ds(0, rem)],
                          acc.at[pl.ds(0, rem)], zsem).wait()
    plsc.subcore_barrier()

    def issue_g(c, x):
        pltpu.async_copy(haug_hbm.at[dst_s.at[c]], rows[x], gsem[x])
        pltpu.async_copy(s1_hbm.at[src_s.at[c]], s1g[x], gsem[x])

    def wait_g(x):
        pltpu.make_async_copy(haug_hbm.at[pl.ds(0, K)], rows[x], gsem[x]).wait()
        pltpu.make_async_copy(s1_hbm.at[pl.ds(0, K)], s1g[x], gsem[x]).wait()

    def wait_s(x):
        pltpu.make_async_copy(rows[x], acc.at[pl.ds(0, K)], ssem[x]).wait()

    def section(c, x, prefetch=None):
        wait_g(x)

        @plsc.parallel_loop(0, K, step=16)
        def mul_body(r0):
            sl0 = pl.ds(r0, 16)
            ridx = r0 + lax.iota(jnp.int32, 16)
            s2v = plsc.load_gather(rows[x], [ridx, jnp.full((16,), NH + 1,
                                                            jnp.int32)])
            lg = s1g[x][sl0] + s2v
            wv16 = jnp.exp(-jnp.maximum(lg, lg * ALPHA))
            for r in range(16):
                wvec = _lane_bcast(wv16, r)
                for j in range(DAUG // 16):
                    sl = pl.ds(j * 16, 16)
                    rows[x][r0 + r, sl] = rows[x][r0 + r, sl] * wvec

        if prefetch is not None:
            z, guard = prefetch
            if guard is None:
                wait_s(z)
                issue_g(c + 2, z)
            else:
                @pl.when(guard[0])
                def _():
                    wait_s(z)

                @pl.when(guard[1])
                def _():
                    issue_g(c + 2, z)
        pltpu.async_copy(rows[x], acc.at[src_s.at[c]], ssem[x], add=True)

    def super_body(u, _):
        pltpu.sync_copy(src_hbm.at[wid].at[pl.ds(u * SCH, SCH)], src_s)
        pltpu.sync_copy(dst_hbm.at[wid].at[pl.ds(u * SCH, SCH)], dst_s)
        issue_g(0, 0)
        issue_g(1, 1)

        def triple(t, _):
            c0 = 3 * t
            section(c0, 0, prefetch=(2, (t > 0, t >= 0)))
            section(c0 + 1, 1, prefetch=(0, None))
            section(c0 + 2, 2, prefetch=(1, (t < NTRI - 1, t < NTRI - 1)))
            return 0

        lax.fori_loop(0, NTRI, triple, 0)
        section(SCH - 1, 0, prefetch=None)
        wait_s(0)
        wait_s(1)
        wait_s(2)
        return 0

    lax.fori_loop(0, NSUP, super_body, 0)

    plsc.subcore_barrier()
    pltpu.sync_copy(acc.at[pl.ds(sid * RPT, RPT)],
                    out_hbm.at[cid].at[pl.ds(sid * RPT, RPT)])


_edge_call = functools.partial(
    pl.kernel,
    out_type=jax.ShapeDtypeStruct((2, N, DAUG), jnp.float32),
    mesh=plsc.VectorSubcoreMesh(core_axis_name="c", subcore_axis_name="s"),
    compiler_params=pltpu.CompilerParams(use_tc_tiling_on_sc=False,
                                         needs_layout_passes=False),
    scratch_types=(
        [pltpu.VMEM((SCH, K), jnp.int32),
         pltpu.VMEM((SCH, K), jnp.int32)]
        + [pltpu.VMEM((K, DAUG), jnp.float32)] * 3
        + [pltpu.VMEM((K,), jnp.float32)] * 3
        + [pltpu.VMEM_SHARED((N, DAUG), jnp.float32)]
        + [pltpu.SemaphoreType.DMA] * 7
    ),
)(_edge_body)


def _final_body(haug_ref, n0_ref, n1_ref, o_ref):
    ns = n0_ref[...] + n1_ref[...]
    hp = ns[:, :NH] / (ns[:, NH:NH + 1] + 1e-16)
    y = haug_ref[:, :NH] - hp
    o_ref[...] = jnp.where(y > 0, y, jnp.exp(y) - 1.0)


_final_call = pl.pallas_call(
    _final_body,
    grid=(N // ROWB,),
    in_specs=[
        pl.BlockSpec((ROWB, DAUG), lambda i: (i, 0)),
        pl.BlockSpec((ROWB, DAUG), lambda i: (i, 0)),
        pl.BlockSpec((ROWB, DAUG), lambda i: (i, 0)),
    ],
    out_specs=pl.BlockSpec((ROWB, NH), lambda i: (i, 0)),
    out_shape=jax.ShapeDtypeStruct((N, NH), jnp.float32),
)


def kernel(x, adj, no_need_param, W, a):
    src = adj[0].reshape(NW, NCH, K)
    dst = adj[1].reshape(NW, NCH, K)
    haug, s12 = _proj_call(x, W, a)
    part = _edge_call(src, dst, s12[0], haug)
    return _final_call(haug, part[0], part[1])
